# baseline (speedup 1.0000x reference)
import jax
import jax.numpy as jnp
from jax import lax
from jax.experimental import pallas as pl
from jax.experimental.pallas import tpu as pltpu
from jax.experimental.pallas import tpu_sc as plsc

VOCAB = 100000
E = 128
S = 20
M = 50
B = 1024

NSEG = B * M
RPS = S + 1
CH = 5
GIDX = CH * RPS
GPAD = 112
NCHUNKS = NSEG // CH
NWORKERS = 32
CPW = NCHUNKS // NWORKERS
SPW = NSEG // NWORKERS
NBUF = 4

EB = E // 16

A_COEF = [(s - 9.5) / 640.0 for s in range(S - 1)]


def _sc_body(idx_hbm, table_hbm, out_hbm, idx_v, *bufs):
    rbufs = bufs[0:NBUF]
    obufs = bufs[NBUF:2 * NBUF]
    gsems = bufs[2 * NBUF:3 * NBUF]
    osems = bufs[3 * NBUF:4 * NBUF]

    wid = lax.axis_index("s") * 2 + lax.axis_index("c")
    chunk0 = wid * CPW
    seg0 = wid * SPW

    pltpu.sync_copy(idx_hbm.at[pl.ds(chunk0, CPW)], idx_v)

    lane = lax.iota(jnp.int32, 16).astype(jnp.float32)
    bvecs = [lane + (eb * 16 - 63.5) for eb in range(EB)]

    def start_gather(it, buf, sem):
        pltpu.make_async_copy(table_hbm.at[idx_v.at[it]], buf, sem).start()

    def wait_gather(buf, sem):
        pltpu.make_async_copy(table_hbm.at[idx_v.at[0]], buf, sem).wait()

    for b in range(NBUF):
        start_gather(b, rbufs[b], gsems[b])

    def body(g, carry):
        for b in range(NBUF):
            it = NBUF * g + b
            rb, ob, gsem, osem = rbufs[b], obufs[b], gsems[b], osems[b]
            wait_gather(rb, gsem)

            @pl.when(it >= NBUF)
            def _():
                pltpu.make_async_copy(
                    ob, out_hbm.at[pl.ds(0, CH * E)], osem).wait()

            for j in range(CH):
                r0 = j * RPS
                for eb in range(EB):
                    sl = pl.ds(eb * 16, 16)
                    acc = A_COEF[0] * rb[r0, sl]
                    for s in range(1, S - 1):
                        acc = acc + A_COEF[s] * rb[r0 + s, sl]
                    ob[pl.ds(j * E + eb * 16, 16)] = (
                        acc * bvecs[eb] + rb[r0 + S - 1, sl] + rb[r0 + S, sl])

            pltpu.make_async_copy(
                ob, out_hbm.at[pl.ds((seg0 + it * CH) * E, CH * E)], osem).start()

            @pl.when(it + NBUF < CPW)
            def _():
                start_gather(it + NBUF, rb, gsem)
        return carry

    lax.fori_loop(0, CPW // NBUF, body, 0)

    for b in range(NBUF):
        pltpu.make_async_copy(
            obufs[b], out_hbm.at[pl.ds(0, CH * E)], osems[b]).wait()


@jax.jit
def kernel(x, embedding_table, temporal_table):
    xi = x.reshape(NSEG, S).astype(jnp.int32)
    te_idx = (jnp.arange(NSEG, dtype=jnp.int32) % M) + VOCAB
    idx = jnp.concatenate([xi, te_idx[:, None]], axis=1)
    idx = idx.reshape(NCHUNKS, GIDX)
    idx = jnp.pad(idx, ((0, 0), (0, GPAD - GIDX)))

    table = jnp.concatenate([embedding_table, temporal_table], axis=0)

    mesh = plsc.VectorSubcoreMesh(core_axis_name="c", subcore_axis_name="s")
    run = pl.kernel(
        _sc_body,
        mesh=mesh,
        out_type=jax.ShapeDtypeStruct((NSEG * E,), jnp.float32),
        scratch_types=(
            [pltpu.VMEM((CPW, GPAD), jnp.int32)]
            + [pltpu.VMEM((GPAD, E), jnp.float32) for _ in range(NBUF)]
            + [pltpu.VMEM((CH * E,), jnp.float32) for _ in range(NBUF)]
            + [pltpu.SemaphoreType.DMA for _ in range(2 * NBUF)]
        ),
    )
    out = run(idx, table)
    return out.reshape(B, M, E)


# speedup vs baseline: 1.4434x; 1.4434x over previous
import jax
import jax.numpy as jnp
from jax import lax
from jax.experimental import pallas as pl
from jax.experimental.pallas import tpu as pltpu
from jax.experimental.pallas import tpu_sc as plsc

VOCAB = 100000
E = 128
S = 20
M = 50
B = 1024

NSEG = B * M
RPS = S + 1
CH = 5
GIDX = CH * RPS
GPAD = 112
NCHUNKS = NSEG // CH
NWORKERS = 32
CPW = NCHUNKS // NWORKERS
SPW = NSEG // NWORKERS
NBUF = 4

EB = E // 16

A_COEF = [(s - 9.5) / 640.0 for s in range(S - 1)]


def _sc_body(idx_hbm, table_hbm, out_hbm, idx_v, *bufs):
    rbufs = bufs[0:NBUF]
    obufs = bufs[NBUF:2 * NBUF]
    gsems = bufs[2 * NBUF:3 * NBUF]
    osems = bufs[3 * NBUF:4 * NBUF]

    wid = lax.axis_index("s") * 2 + lax.axis_index("c")
    chunk0 = wid * CPW
    seg0 = wid * SPW

    pltpu.sync_copy(idx_hbm.at[pl.ds(chunk0, CPW)], idx_v)

    lane = lax.iota(jnp.int32, 16).astype(jnp.float32)
    bvecs = [lane + (eb * 16 - 63.5) for eb in range(EB)]

    def start_gather(it, buf, sem):
        pltpu.make_async_copy(table_hbm.at[idx_v.at[it]], buf, sem).start()

    def wait_gather(buf, sem):
        pltpu.make_async_copy(table_hbm.at[idx_v.at[0]], buf, sem).wait()

    for b in range(NBUF):
        start_gather(b, rbufs[b], gsems[b])

    def body(g, carry):
        for b in range(NBUF):
            it = NBUF * g + b
            rb, ob, gsem, osem = rbufs[b], obufs[b], gsems[b], osems[b]
            wait_gather(rb, gsem)

            @pl.when(it >= NBUF)
            def _():
                pltpu.make_async_copy(
                    ob, out_hbm.at[pl.ds(0, CH * E)], osem).wait()

            for j in range(CH):
                r0 = j * RPS
                for eb2 in range(EB // 2):
                    sl = pl.ds(eb2 * 16, 16)

                    def rowpair(r):
                        w = rb[r, sl]
                        lo = plsc.bitcast(w << 16, jnp.float32)
                        hi = plsc.bitcast(w & jnp.int32(-65536), jnp.float32)
                        return lo, hi

                    lo, hi = rowpair(r0)
                    acc_lo = A_COEF[0] * lo
                    acc_hi = A_COEF[0] * hi
                    for s in range(1, S - 1):
                        lo, hi = rowpair(r0 + s)
                        acc_lo = acc_lo + A_COEF[s] * lo
                        acc_hi = acc_hi + A_COEF[s] * hi
                    l19, h19 = rowpair(r0 + S - 1)
                    l20, h20 = rowpair(r0 + S)
                    ob[pl.ds(j * E + eb2 * 32, 16)] = (
                        acc_lo * bvecs[2 * eb2] + l19 + l20)
                    ob[pl.ds(j * E + eb2 * 32 + 16, 16)] = (
                        acc_hi * bvecs[2 * eb2 + 1] + h19 + h20)

            pltpu.make_async_copy(
                ob, out_hbm.at[pl.ds((seg0 + it * CH) * E, CH * E)], osem).start()

            @pl.when(it + NBUF < CPW)
            def _():
                start_gather(it + NBUF, rb, gsem)
        return carry

    lax.fori_loop(0, CPW // NBUF, body, 0)

    for b in range(NBUF):
        pltpu.make_async_copy(
            obufs[b], out_hbm.at[pl.ds(0, CH * E)], osems[b]).wait()


@jax.jit
def kernel(x, embedding_table, temporal_table):
    xi = x.reshape(NSEG, S).astype(jnp.int32)
    te_idx = (jnp.arange(NSEG, dtype=jnp.int32) % M) + VOCAB
    idx = jnp.concatenate([xi, te_idx[:, None]], axis=1)
    idx = idx.reshape(NCHUNKS, GIDX)
    idx = jnp.pad(idx, ((0, 0), (0, GPAD - GIDX)))

    table = jnp.concatenate([embedding_table, temporal_table], axis=0)
    c = jnp.arange(E)
    perm = 32 * (c // 32) + (c % 2) * 16 + (c % 32) // 2
    table = table[:, perm].astype(jnp.bfloat16)
    table = jax.lax.bitcast_convert_type(
        table.reshape(VOCAB + M, E // 2, 2), jnp.int32)

    mesh = plsc.VectorSubcoreMesh(core_axis_name="c", subcore_axis_name="s")
    run = pl.kernel(
        _sc_body,
        mesh=mesh,
        compiler_params=pltpu.CompilerParams(
            needs_layout_passes=False, use_tc_tiling_on_sc=False),
        out_type=jax.ShapeDtypeStruct((NSEG * E,), jnp.float32),
        scratch_types=(
            [pltpu.VMEM((CPW, GPAD), jnp.int32)]
            + [pltpu.VMEM((GPAD, E // 2), jnp.int32) for _ in range(NBUF)]
            + [pltpu.VMEM((CH * E,), jnp.float32) for _ in range(NBUF)]
            + [pltpu.SemaphoreType.DMA for _ in range(2 * NBUF)]
        ),
    )
    out = run(idx, table)
    return out.reshape(B, M, E)


# speedup vs baseline: 2.0165x; 1.3970x over previous
import jax
import jax.numpy as jnp
from jax import lax
from jax.experimental import pallas as pl
from jax.experimental.pallas import tpu as pltpu
from jax.experimental.pallas import tpu_sc as plsc

VOCAB = 100000
E = 128
S = 20
M = 50
B = 1024

NSEG = B * M
RPS = S
CH = 5
GIDX = CH * RPS
GPAD = 104
NCHUNKS = NSEG // CH
NWORKERS = 32
CPW = NCHUNKS // NWORKERS
SPW = NSEG // NWORKERS
NBUF = 4

EB = E // 16

A_COEF = [(s - 9.5) / 640.0 for s in range(S - 1)]


def _sc_body(idx_hbm, table_hbm, te_hbm, out_hbm, idx_v, te_v, *bufs):
    rbufs = bufs[0:NBUF]
    obufs = bufs[NBUF:2 * NBUF]
    gsems = bufs[2 * NBUF:3 * NBUF]
    osems = bufs[3 * NBUF:4 * NBUF]

    wid = lax.axis_index("s") * 2 + lax.axis_index("c")
    chunk0 = wid * CPW
    seg0 = wid * SPW

    pltpu.sync_copy(idx_hbm.at[pl.ds(chunk0, CPW)], idx_v)
    pltpu.sync_copy(te_hbm, te_v)

    lane = lax.iota(jnp.int32, 16).astype(jnp.float32)
    bvecs = [lane + (eb * 16 - 63.5) for eb in range(EB)]

    def start_gather(it, buf, sem):
        pltpu.make_async_copy(table_hbm.at[idx_v.at[it]], buf, sem).start()

    def wait_gather(buf, sem):
        pltpu.make_async_copy(table_hbm.at[idx_v.at[0]], buf, sem).wait()

    for b in range(NBUF):
        start_gather(b, rbufs[b], gsems[b])

    def body(g, carry):
        for b in range(NBUF):
            it = NBUF * g + b
            rb, ob, gsem, osem = rbufs[b], obufs[b], gsems[b], osems[b]
            wait_gather(rb, gsem)

            @pl.when(it >= NBUF)
            def _():
                pltpu.make_async_copy(
                    ob, out_hbm.at[pl.ds(0, CH * E)], osem).wait()

            mrow = (it * CH) % M

            for j in range(CH):
                r0 = j * RPS
                for eb2 in range(EB // 2):
                    sl = pl.ds(eb2 * 16, 16)

                    def rowpair(r):
                        w = rb[r, sl]
                        lo = plsc.bitcast(w << 16, jnp.float32)
                        hi = plsc.bitcast(w & jnp.int32(-65536), jnp.float32)
                        return lo, hi

                    lo, hi = rowpair(r0)
                    acc_lo = A_COEF[0] * lo
                    acc_hi = A_COEF[0] * hi
                    for s in range(1, S - 1):
                        lo, hi = rowpair(r0 + s)
                        acc_lo = acc_lo + A_COEF[s] * lo
                        acc_hi = acc_hi + A_COEF[s] * hi
                    l19, h19 = rowpair(r0 + S - 1)
                    tw = te_v[pl.ds((mrow + j) * (E // 2) + eb2 * 16, 16)]
                    tlo = plsc.bitcast(tw << 16, jnp.float32)
                    thi = plsc.bitcast(tw & jnp.int32(-65536), jnp.float32)
                    ob[pl.ds(j * E + eb2 * 32, 16)] = (
                        acc_lo * bvecs[2 * eb2] + l19 + tlo)
                    ob[pl.ds(j * E + eb2 * 32 + 16, 16)] = (
                        acc_hi * bvecs[2 * eb2 + 1] + h19 + thi)

            pltpu.make_async_copy(
                ob, out_hbm.at[pl.ds((seg0 + it * CH) * E, CH * E)], osem).start()

            @pl.when(it + NBUF < CPW)
            def _():
                start_gather(it + NBUF, rb, gsem)
        return carry

    lax.fori_loop(0, CPW // NBUF, body, 0)

    for b in range(NBUF):
        pltpu.make_async_copy(
            obufs[b], out_hbm.at[pl.ds(0, CH * E)], osems[b]).wait()


@jax.jit
def kernel(x, embedding_table, temporal_table):
    xi = x.reshape(NSEG, S).astype(jnp.int32)
    idx = xi.reshape(NCHUNKS, GIDX)
    idx = jnp.pad(idx, ((0, 0), (0, GPAD - GIDX)))

    table = embedding_table
    c = jnp.arange(E)
    perm = 32 * (c // 32) + (c % 2) * 16 + (c % 32) // 2
    table = table[:, perm].astype(jnp.bfloat16)
    table = jax.lax.bitcast_convert_type(
        table.reshape(VOCAB, E // 2, 2), jnp.int32)
    te = temporal_table[:, perm].astype(jnp.bfloat16)
    te = jax.lax.bitcast_convert_type(
        te.reshape(M, E // 2, 2), jnp.int32).reshape(M * E // 2)

    mesh = plsc.VectorSubcoreMesh(core_axis_name="c", subcore_axis_name="s")
    run = pl.kernel(
        _sc_body,
        mesh=mesh,
        compiler_params=pltpu.CompilerParams(
            needs_layout_passes=False, use_tc_tiling_on_sc=False),
        out_type=jax.ShapeDtypeStruct((NSEG * E,), jnp.float32),
        scratch_types=(
            [pltpu.VMEM((CPW, GPAD), jnp.int32),
             pltpu.VMEM((M * E // 2,), jnp.int32)]
            + [pltpu.VMEM((GPAD, E // 2), jnp.int32) for _ in range(NBUF)]
            + [pltpu.VMEM((CH * E,), jnp.float32) for _ in range(NBUF)]
            + [pltpu.SemaphoreType.DMA for _ in range(2 * NBUF)]
        ),
    )
    out = run(idx, table, te)
    return out.reshape(B, M, E)


# speedup vs baseline: 3.0480x; 1.5116x over previous
import jax
import jax.numpy as jnp
from jax import lax
from jax.experimental import pallas as pl
from jax.experimental.pallas import tpu as pltpu
from jax.experimental.pallas import tpu_sc as plsc

VOCAB = 100000
E = 128
S = 20
M = 50
B = 1024

NSEG = B * M
RPS = S
CH = 4
GIDX = CH * RPS
GPAD = 80
NCHUNKS = NSEG // CH
NWORKERS = 32
CPW = NCHUNKS // NWORKERS
SPW = NSEG // NWORKERS
NBUF = 4

EB = E // 16

A_COEF = [(s - 9.5) / 640.0 for s in range(S - 1)]


def _sc_body(idx_hbm, table_hbm, te_hbm, out_hbm, idx_v, te_v, *bufs):
    rbufs = bufs[0:NBUF]
    obufs = bufs[NBUF:2 * NBUF]
    gsems = bufs[2 * NBUF:3 * NBUF]
    osems = bufs[3 * NBUF:4 * NBUF]

    wid = lax.axis_index("s") * 2 + lax.axis_index("c")
    chunk0 = wid * CPW
    seg0 = wid * SPW

    pltpu.sync_copy(idx_hbm.at[pl.ds(chunk0, CPW)], idx_v)
    pltpu.sync_copy(te_hbm, te_v)

    lane = lax.iota(jnp.int32, 16).astype(jnp.float32)
    bvecs = [lane + (eb * 16 - 63.5) for eb in range(EB)]

    def start_gather(it, buf, sem):
        pltpu.make_async_copy(table_hbm.at[idx_v.at[it]], buf, sem).start()

    def wait_gather(buf, sem):
        pltpu.make_async_copy(table_hbm.at[idx_v.at[0]], buf, sem).wait()

    for b in range(NBUF):
        start_gather(b, rbufs[b], gsems[b])

    def body(g, carry):
        for b in range(NBUF):
            it = NBUF * g + b
            rb, ob, gsem, osem = rbufs[b], obufs[b], gsems[b], osems[b]
            wait_gather(rb, gsem)

            @pl.when(it >= NBUF)
            def _():
                pltpu.make_async_copy(
                    ob, out_hbm.at[pl.ds(0, CH * E)], osem).wait()

            mrow = (it * CH) % M

            for j in range(CH):
                r0 = j * RPS
                for eb2 in range(EB // 2):
                    sl = pl.ds(eb2 * 16, 16)

                    def rowpair(r):
                        w = rb[r, sl]
                        lo = plsc.bitcast(w << 16, jnp.float32)
                        hi = plsc.bitcast(w & jnp.int32(-65536), jnp.float32)
                        return lo, hi

                    lo, hi = rowpair(r0)
                    acc_lo = A_COEF[0] * lo
                    acc_hi = A_COEF[0] * hi
                    for s in range(1, S - 1):
                        lo, hi = rowpair(r0 + s)
                        acc_lo = acc_lo + A_COEF[s] * lo
                        acc_hi = acc_hi + A_COEF[s] * hi
                    l19, h19 = rowpair(r0 + S - 1)
                    tw = te_v[pl.ds((mrow + j) * (E // 2) + eb2 * 16, 16)]
                    tlo = plsc.bitcast(tw << 16, jnp.float32)
                    thi = plsc.bitcast(tw & jnp.int32(-65536), jnp.float32)
                    ob[pl.ds(j * E + eb2 * 32, 16)] = (
                        acc_lo * bvecs[2 * eb2] + l19 + tlo)
                    ob[pl.ds(j * E + eb2 * 32 + 16, 16)] = (
                        acc_hi * bvecs[2 * eb2 + 1] + h19 + thi)

            pltpu.make_async_copy(
                ob, out_hbm.at[pl.ds((seg0 + it * CH) * E, CH * E)], osem).start()

            @pl.when(it + NBUF < CPW)
            def _():
                start_gather(it + NBUF, rb, gsem)
        return carry

    lax.fori_loop(0, CPW // NBUF, body, 0)

    for b in range(NBUF):
        pltpu.make_async_copy(
            obufs[b], out_hbm.at[pl.ds(0, CH * E)], osems[b]).wait()


@jax.jit
def kernel(x, embedding_table, temporal_table):
    idx = x.astype(jnp.int32).reshape(NCHUNKS, GIDX)

    table = embedding_table
    c = jnp.arange(E)
    perm = 32 * (c // 32) + (c % 2) * 16 + (c % 32) // 2
    table = table[:, perm].astype(jnp.bfloat16)
    table = jax.lax.bitcast_convert_type(
        table.reshape(VOCAB, E // 2, 2), jnp.int32)
    te = temporal_table[:, perm].astype(jnp.bfloat16)
    te = jax.lax.bitcast_convert_type(
        te.reshape(M, E // 2, 2), jnp.int32).reshape(M * E // 2)

    mesh = plsc.VectorSubcoreMesh(core_axis_name="c", subcore_axis_name="s")
    run = pl.kernel(
        _sc_body,
        mesh=mesh,
        compiler_params=pltpu.CompilerParams(
            needs_layout_passes=False, use_tc_tiling_on_sc=False),
        out_type=jax.ShapeDtypeStruct((NSEG * E,), jnp.float32),
        scratch_types=(
            [pltpu.VMEM((CPW, GPAD), jnp.int32),
             pltpu.VMEM((M * E // 2,), jnp.int32)]
            + [pltpu.VMEM((GPAD, E // 2), jnp.int32) for _ in range(NBUF)]
            + [pltpu.VMEM((CH * E,), jnp.float32) for _ in range(NBUF)]
            + [pltpu.SemaphoreType.DMA for _ in range(2 * NBUF)]
        ),
    )
    out = run(idx, table, te)
    return out.reshape(B, M, E)


# speedup vs baseline: 3.0539x; 1.0019x over previous
import jax
import jax.numpy as jnp
from jax import lax
from jax.experimental import pallas as pl
from jax.experimental.pallas import tpu as pltpu
from jax.experimental.pallas import tpu_sc as plsc

VOCAB = 100000
E = 128
S = 20
M = 50
B = 1024

NSEG = B * M
RPS = S
CH = 4
GIDX = CH * RPS
GPAD = 80
NCHUNKS = NSEG // CH
NWORKERS = 32
CPW = NCHUNKS // NWORKERS
SPW = NSEG // NWORKERS
NBUF = 4

EB = E // 16

A_COEF = [(s - 9.5) / 640.0 for s in range(S - 1)]


def _sc_body(idx_hbm, table_hbm, te_hbm, out_hbm, idx_v, te_v, *bufs):
    rbufs = bufs[0:NBUF]
    obufs = bufs[NBUF:2 * NBUF]
    gsems = bufs[2 * NBUF:3 * NBUF]
    osems = bufs[3 * NBUF:4 * NBUF]

    wid = lax.axis_index("s") * 2 + lax.axis_index("c")
    chunk0 = wid * CPW
    seg0 = wid * SPW

    pltpu.sync_copy(idx_hbm.at[pl.ds(chunk0, CPW)], idx_v)
    pltpu.sync_copy(te_hbm, te_v)

    lane = lax.iota(jnp.int32, 16).astype(jnp.float32)
    bvecs = [lane + (eb * 16 - 63.5) for eb in range(EB)]

    def start_gather(it, buf, sem):
        pltpu.make_async_copy(table_hbm.at[idx_v.at[it]], buf, sem).start()

    def wait_gather(buf, sem):
        pltpu.make_async_copy(table_hbm.at[idx_v.at[0]], buf, sem).wait()

    for b in range(NBUF):
        start_gather(b, rbufs[b], gsems[b])

    def body(g, carry):
        for b in range(NBUF):
            it = NBUF * g + b
            rb, ob, gsem, osem = rbufs[b], obufs[b], gsems[b], osems[b]
            wait_gather(rb, gsem)

            @pl.when(it >= NBUF)
            def _():
                pltpu.make_async_copy(
                    ob, out_hbm.at[pl.ds(0, CH * E)], osem).wait()

            seg_in_batch = it * CH

            for j in range(CH):
                r0 = j * RPS
                mj = lax.rem(seg_in_batch + j, M)
                for eb2 in range(EB // 2):
                    sl = pl.ds(eb2 * 16, 16)

                    def rowpair(r):
                        w = rb[r, sl]
                        lo = plsc.bitcast(w << 16, jnp.float32)
                        hi = plsc.bitcast(w & jnp.int32(-65536), jnp.float32)
                        return lo, hi

                    lo, hi = rowpair(r0)
                    acc_lo = A_COEF[0] * lo
                    acc_hi = A_COEF[0] * hi
                    for s in range(1, S - 1):
                        lo, hi = rowpair(r0 + s)
                        acc_lo = acc_lo + A_COEF[s] * lo
                        acc_hi = acc_hi + A_COEF[s] * hi
                    l19, h19 = rowpair(r0 + S - 1)
                    tw = te_v[pl.ds(mj * (E // 2) + eb2 * 16, 16)]
                    tlo = plsc.bitcast(tw << 16, jnp.float32)
                    thi = plsc.bitcast(tw & jnp.int32(-65536), jnp.float32)
                    ob[pl.ds(j * E + eb2 * 32, 16)] = (
                        acc_lo * bvecs[2 * eb2] + l19 + tlo)
                    ob[pl.ds(j * E + eb2 * 32 + 16, 16)] = (
                        acc_hi * bvecs[2 * eb2 + 1] + h19 + thi)

            pltpu.make_async_copy(
                ob, out_hbm.at[pl.ds((seg0 + it * CH) * E, CH * E)], osem).start()

            @pl.when(it + NBUF < CPW)
            def _():
                start_gather(it + NBUF, rb, gsem)
        return carry

    lax.fori_loop(0, CPW // NBUF, body, 0)

    for b in range(NBUF):
        pltpu.make_async_copy(
            obufs[b], out_hbm.at[pl.ds(0, CH * E)], osems[b]).wait()


@jax.jit
def kernel(x, embedding_table, temporal_table):
    idx = x.astype(jnp.int32).reshape(NCHUNKS, GIDX)

    table = embedding_table
    c = jnp.arange(E)
    perm = 32 * (c // 32) + (c % 2) * 16 + (c % 32) // 2
    table = table[:, perm].astype(jnp.bfloat16)
    table = jax.lax.bitcast_convert_type(
        table.reshape(VOCAB, E // 2, 2), jnp.int32)
    te = temporal_table[:, perm].astype(jnp.bfloat16)
    te = jax.lax.bitcast_convert_type(
        te.reshape(M, E // 2, 2), jnp.int32).reshape(M * E // 2)

    mesh = plsc.VectorSubcoreMesh(core_axis_name="c", subcore_axis_name="s")
    run = pl.kernel(
        _sc_body,
        mesh=mesh,
        compiler_params=pltpu.CompilerParams(
            needs_layout_passes=False, use_tc_tiling_on_sc=False),
        out_type=jax.ShapeDtypeStruct((NSEG * E,), jnp.float32),
        scratch_types=(
            [pltpu.VMEM((CPW, GPAD), jnp.int32),
             pltpu.VMEM((M * E // 2,), jnp.int32)]
            + [pltpu.VMEM((GPAD, E // 2), jnp.int32) for _ in range(NBUF)]
            + [pltpu.VMEM((CH * E,), jnp.float32) for _ in range(NBUF)]
            + [pltpu.SemaphoreType.DMA for _ in range(2 * NBUF)]
        ),
    )
    out = run(idx, table, te)
    return out.reshape(B, M, E)


# speedup vs baseline: 4.2723x; 1.3990x over previous
import jax
import jax.numpy as jnp
from jax import lax
from jax.experimental import pallas as pl
from jax.experimental.pallas import tpu as pltpu
from jax.experimental.pallas import tpu_sc as plsc

VOCAB = 100000
E = 128
S = 20
M = 50
B = 1024

NSEG = B * M
RPS = S
CH = 4
GIDX = CH * RPS
GPAD = 80
NCHUNKS = NSEG // CH
NWORKERS = 32
CPW = NCHUNKS // NWORKERS
SPW = NSEG // NWORKERS
NBUF = 4

EB = E // 16

A_COEF = [(s - 9.5) / 640.0 for s in range(S - 1)]


def _sc_body(idx_hbm, table_hbm, te_hbm, out_hbm, idx_v, te_v, *bufs):
    rbufs = bufs[0:NBUF]
    obufs = bufs[NBUF:2 * NBUF]
    gsems = bufs[2 * NBUF:3 * NBUF]
    osems = bufs[3 * NBUF:4 * NBUF]

    wid = lax.axis_index("s") * 2 + lax.axis_index("c")
    chunk0 = wid * CPW
    seg0 = wid * SPW

    pltpu.sync_copy(idx_hbm.at[pl.ds(chunk0, CPW)], idx_v)
    pltpu.sync_copy(te_hbm, te_v)

    lane = lax.iota(jnp.int32, 16).astype(jnp.float32)
    bv_even = [2.0 * lane + (32 * k - 63.5) for k in range(EB // 2)]
    bv_odd = [2.0 * lane + (32 * k + 1 - 63.5) for k in range(EB // 2)]
    ilane2 = lax.iota(jnp.int32, 16) * 2

    def start_gather(it, buf, sem):
        pltpu.make_async_copy(table_hbm.at[idx_v.at[it]], buf, sem).start()

    def wait_gather(buf, sem):
        pltpu.make_async_copy(table_hbm.at[idx_v.at[0]], buf, sem).wait()

    for b in range(NBUF):
        start_gather(b, rbufs[b], gsems[b])

    def body(g, carry):
        for b in range(NBUF):
            it = NBUF * g + b
            rb, ob, gsem, osem = rbufs[b], obufs[b], gsems[b], osems[b]
            wait_gather(rb, gsem)

            @pl.when(it >= NBUF)
            def _():
                pltpu.make_async_copy(
                    ob, out_hbm.at[pl.ds(0, CH * E)], osem).wait()

            seg_in_batch = it * CH

            for j in range(CH):
                r0 = j * RPS
                mj = lax.rem(seg_in_batch + j, M)
                for eb2 in range(EB // 2):
                    sl = pl.ds(eb2 * 32, 32)

                    def rowpair(r):
                        w = plsc.bitcast(rb[r, sl], jnp.int32)
                        lo = plsc.bitcast(w << 16, jnp.float32)
                        hi = plsc.bitcast(w & jnp.int32(-65536), jnp.float32)
                        return lo, hi

                    lo, hi = rowpair(r0)
                    acc_lo = A_COEF[0] * lo
                    acc_hi = A_COEF[0] * hi
                    for s in range(1, S - 1):
                        lo, hi = rowpair(r0 + s)
                        acc_lo = acc_lo + A_COEF[s] * lo
                        acc_hi = acc_hi + A_COEF[s] * hi
                    l19, h19 = rowpair(r0 + S - 1)
                    tw = plsc.bitcast(
                        te_v[pl.ds(mj * E + eb2 * 32, 32)], jnp.int32)
                    tlo = plsc.bitcast(tw << 16, jnp.float32)
                    thi = plsc.bitcast(tw & jnp.int32(-65536), jnp.float32)
                    iv = ilane2 + (j * E + eb2 * 32)
                    plsc.store_scatter(
                        ob, [iv], acc_lo * bv_even[eb2] + l19 + tlo)
                    plsc.store_scatter(
                        ob, [iv + 1], acc_hi * bv_odd[eb2] + h19 + thi)

            pltpu.make_async_copy(
                ob, out_hbm.at[pl.ds((seg0 + it * CH) * E, CH * E)], osem).start()

            @pl.when(it + NBUF < CPW)
            def _():
                start_gather(it + NBUF, rb, gsem)
        return carry

    lax.fori_loop(0, CPW // NBUF, body, 0)

    for b in range(NBUF):
        pltpu.make_async_copy(
            obufs[b], out_hbm.at[pl.ds(0, CH * E)], osems[b]).wait()


@jax.jit
def kernel(x, embedding_table, temporal_table):
    idx = x.astype(jnp.int32).reshape(NCHUNKS, GIDX)

    table = embedding_table.astype(jnp.bfloat16)
    te = temporal_table.astype(jnp.bfloat16).reshape(M * E)

    mesh = plsc.VectorSubcoreMesh(core_axis_name="c", subcore_axis_name="s")
    run = pl.kernel(
        _sc_body,
        mesh=mesh,
        compiler_params=pltpu.CompilerParams(
            needs_layout_passes=False, use_tc_tiling_on_sc=False),
        out_type=jax.ShapeDtypeStruct((NSEG * E,), jnp.float32),
        scratch_types=(
            [pltpu.VMEM((CPW, GPAD), jnp.int32),
             pltpu.VMEM((M * E,), jnp.bfloat16)]
            + [pltpu.VMEM((GPAD, E), jnp.bfloat16) for _ in range(NBUF)]
            + [pltpu.VMEM((CH * E,), jnp.float32) for _ in range(NBUF)]
            + [pltpu.SemaphoreType.DMA for _ in range(2 * NBUF)]
        ),
    )
    out = run(idx, table, te)
    return out.reshape(B, M, E)


# speedup vs baseline: 4.2974x; 1.0059x over previous
import jax
import jax.numpy as jnp
from jax import lax
from jax.experimental import pallas as pl
from jax.experimental.pallas import tpu as pltpu
from jax.experimental.pallas import tpu_sc as plsc

VOCAB = 100000
E = 128
S = 20
M = 50
B = 1024

NSEG = B * M
RPS = S
CH = 4
GIDX = CH * RPS
GPAD = 80
NCHUNKS = NSEG // CH
NWORKERS = 32
CPW = NCHUNKS // NWORKERS
SPW = NSEG // NWORKERS
NBUF = 4

EB = E // 16

A_COEF = [(s - 9.5) / 640.0 for s in range(S - 1)]


def _sc_body(idx_hbm, table_hbm, te_hbm, out_hbm, idx_v, te_v, *bufs):
    rbufs = bufs[0:NBUF]
    obufs = bufs[NBUF:2 * NBUF]
    gsems = bufs[2 * NBUF:3 * NBUF]
    osems = bufs[3 * NBUF:4 * NBUF]

    wid = lax.axis_index("s") * 2 + lax.axis_index("c")
    chunk0 = wid * CPW
    seg0 = wid * SPW

    pltpu.sync_copy(idx_hbm.at[pl.ds(chunk0, CPW)], idx_v)
    pltpu.sync_copy(te_hbm, te_v)

    lane = lax.iota(jnp.int32, 16).astype(jnp.float32)
    bv_even = [2.0 * lane + (32 * k - 63.5) for k in range(EB // 2)]
    bv_odd = [2.0 * lane + (32 * k + 1 - 63.5) for k in range(EB // 2)]
    ilane2 = lax.iota(jnp.int32, 16) * 2
    zlane = lax.iota(jnp.int32, 16) * 0
    jvecs = [zlane + j for j in range(CH)]

    def start_gather(it, buf, sem):
        pltpu.make_async_copy(table_hbm.at[idx_v.at[it]], buf, sem).start()

    def wait_gather(buf, sem):
        pltpu.make_async_copy(table_hbm.at[idx_v.at[0]], buf, sem).wait()

    for b in range(NBUF):
        start_gather(b, rbufs[b], gsems[b])

    def body(g, carry):
        for b in range(NBUF):
            it = NBUF * g + b
            rb, ob, gsem, osem = rbufs[b], obufs[b], gsems[b], osems[b]
            wait_gather(rb, gsem)

            @pl.when(it >= NBUF)
            def _():
                pltpu.make_async_copy(
                    ob, out_hbm.at[pl.ds(0, CH)], osem).wait()

            seg_in_batch = it * CH

            for j in range(CH):
                r0 = j * RPS
                mj = lax.rem(seg_in_batch + j, M)
                for eb2 in range(EB // 2):
                    sl = pl.ds(eb2 * 32, 32)

                    def rowpair(r):
                        w = plsc.bitcast(rb[r, sl], jnp.int32)
                        lo = plsc.bitcast(w << 16, jnp.float32)
                        hi = plsc.bitcast(w & jnp.int32(-65536), jnp.float32)
                        return lo, hi

                    lo, hi = rowpair(r0)
                    acc_lo = A_COEF[0] * lo
                    acc_hi = A_COEF[0] * hi
                    for s in range(1, S - 1):
                        lo, hi = rowpair(r0 + s)
                        acc_lo = acc_lo + A_COEF[s] * lo
                        acc_hi = acc_hi + A_COEF[s] * hi
                    l19, h19 = rowpair(r0 + S - 1)
                    tw = plsc.bitcast(
                        te_v[pl.ds(mj * E + eb2 * 32, 32)], jnp.int32)
                    tlo = plsc.bitcast(tw << 16, jnp.float32)
                    thi = plsc.bitcast(tw & jnp.int32(-65536), jnp.float32)
                    iv = ilane2 + (eb2 * 32)
                    jv = jvecs[j]
                    plsc.store_scatter(
                        ob, [jv, iv], acc_lo * bv_even[eb2] + l19 + tlo)
                    plsc.store_scatter(
                        ob, [jv, iv + 1], acc_hi * bv_odd[eb2] + h19 + thi)

            pltpu.make_async_copy(
                ob, out_hbm.at[pl.ds(seg0 + it * CH, CH)], osem).start()

            @pl.when(it + NBUF < CPW)
            def _():
                start_gather(it + NBUF, rb, gsem)
        return carry

    lax.fori_loop(0, CPW // NBUF, body, 0)

    for b in range(NBUF):
        pltpu.make_async_copy(
            obufs[b], out_hbm.at[pl.ds(0, CH)], osems[b]).wait()


@jax.jit
def kernel(x, embedding_table, temporal_table):
    idx = x.astype(jnp.int32).reshape(NCHUNKS, GIDX)

    table = embedding_table.astype(jnp.bfloat16)
    te = temporal_table.astype(jnp.bfloat16).reshape(M * E)

    mesh = plsc.VectorSubcoreMesh(core_axis_name="c", subcore_axis_name="s")
    run = pl.kernel(
        _sc_body,
        mesh=mesh,
        compiler_params=pltpu.CompilerParams(
            needs_layout_passes=False, use_tc_tiling_on_sc=False),
        out_type=jax.ShapeDtypeStruct((NSEG, E), jnp.float32),
        scratch_types=(
            [pltpu.VMEM((CPW, GPAD), jnp.int32),
             pltpu.VMEM((M * E,), jnp.bfloat16)]
            + [pltpu.VMEM((GPAD, E), jnp.bfloat16) for _ in range(NBUF)]
            + [pltpu.VMEM((CH, E), jnp.float32) for _ in range(NBUF)]
            + [pltpu.SemaphoreType.DMA for _ in range(2 * NBUF)]
        ),
    )
    out = run(idx, table, te)
    return out.reshape(B, M, E)
